# baseline (speedup 1.0000x reference)
import functools

import jax
import jax.numpy as jnp
from jax import lax
from jax.experimental import pallas as pl
from jax.experimental.pallas import tpu as pltpu
from jax.experimental.pallas import tpu_sc as plsc

F32 = jnp.float32
BF16 = jnp.bfloat16
XP = 16
BE = 2560
BN = 2000
CH = 400
CXD = 200


def _silu(v):
    return v * jax.nn.sigmoid(v)


def _init_body(h_ref, we_ref, be_ref, w1a_ref, w1b_ref, b1_ref,
               wee_ref, w1e_ref, bee_ref,
               h0_ref, hw1_ref, hw2_ref, wfea_ref, bfea_ref):
    h0 = jnp.dot(h_ref[...], we_ref[...], preferred_element_type=F32) + be_ref[...]
    h0_ref[...] = h0
    hw1_ref[...] = jnp.dot(h0, w1a_ref[...], preferred_element_type=F32) + b1_ref[...]
    hw2_ref[...] = jnp.dot(h0, w1b_ref[...], preferred_element_type=F32)
    wfea_ref[...] = jnp.dot(wee_ref[...], w1e_ref[...], preferred_element_type=F32)
    bfea_ref[...] = jnp.dot(bee_ref[...], w1e_ref[...], preferred_element_type=F32)


def _tc_init(h, we, be, w1a, w1b, b1, wee, w1e, bee, n, hid):
    grid = (n // BN,)
    blk = lambda r, c: pl.BlockSpec((r, c), lambda i: (i, 0))
    const = lambda r, c: pl.BlockSpec((r, c), lambda i: (0, 0))
    return pl.pallas_call(
        _init_body,
        grid=grid,
        in_specs=[blk(BN, hid), const(hid, hid), const(1, hid),
                  const(hid, hid), const(hid, hid), const(1, hid),
                  const(16, hid), const(hid, hid), const(1, hid)],
        out_specs=[blk(BN, hid), blk(BN, hid), blk(BN, hid),
                   const(16, hid), const(1, hid)],
        out_shape=[jax.ShapeDtypeStruct((n, hid), F32),
                   jax.ShapeDtypeStruct((n, hid), F32),
                   jax.ShapeDtypeStruct((n, hid), F32),
                   jax.ShapeDtypeStruct((16, hid), F32),
                   jax.ShapeDtypeStruct((1, hid), F32)],
    )(h, we, be, w1a, w1b, b1, wee, w1e, bee)


def _edge_body(first, last, *refs):
    n_wts = 4 + (9 if first else 8) + (2 if last else 0)
    ins, outs = refs[:n_wts], refs[n_wts:]
    ghr_ref, ghc_ref, e_ref, dxy_ref = ins[:4]
    w = list(ins[4:])
    if first:
        w1e_ref, bfea_ref = w[0], w[1]
        w = w[2:]
    else:
        w1e_ref = w[0]
        w = w[1:]
    wr_ref, w2_ref, b2_ref, wc1_ref, bc1_ref, wc2_ref, bc2_ref = w[:7]
    hid = ghr_ref.shape[1]
    be8 = dxy_ref.shape[0]
    dp = dxy_ref[...]
    li = lax.broadcasted_iota(jnp.int32, (hid, hid), 0)
    lj = lax.broadcasted_iota(jnp.int32, (hid, hid), 1)
    gb = (li // XP == lj // XP).astype(F32)
    rp = jnp.dot(dp * dp, gb, preferred_element_type=F32)
    s_a = lax.broadcasted_iota(jnp.int32, (hid, 8), 0)
    s_g = lax.broadcasted_iota(jnp.int32, (hid, 8), 1)
    sel = (s_a == s_g * XP).astype(F32)
    r8 = jnp.dot(rp, sel, preferred_element_type=F32)
    ne = 8 * be8
    a_j = lax.broadcasted_iota(jnp.int32, (ne, 8), 0)
    a_g = lax.broadcasted_iota(jnp.int32, (ne, 8), 1)
    a8 = (a_j % 8 == a_g).astype(F32)
    rrep = lax.broadcast_in_dim(r8, (be8, 8, 8), (0, 2)).reshape(ne, 8)
    radial = jnp.sum(rrep * a8, axis=1, keepdims=True)
    z = (ghr_ref[...] + ghc_ref[...]
         + radial * wr_ref[...]
         + jnp.dot(e_ref[...], w1e_ref[...], preferred_element_type=F32))
    if first:
        z = z + bfea_ref[...]
    m1 = _silu(z)
    m = _silu(jnp.dot(m1, w2_ref[...], preferred_element_type=F32) + b2_ref[...])
    c1 = _silu(jnp.dot(m, wc1_ref[...], preferred_element_type=F32) + bc1_ref[...])
    phi = jnp.sum(c1 * wc2_ref[...], axis=1, keepdims=True) + bc2_ref[:, :1]
    b_g = lax.broadcasted_iota(jnp.int32, (8, hid), 0)
    b_l = lax.broadcasted_iota(jnp.int32, (8, hid), 1)
    b8 = (b_l // XP == b_g).astype(F32)
    phi8 = jnp.sum((phi * a8).reshape(be8, 8, 8), axis=1)
    phip = jnp.dot(phi8, b8, preferred_element_type=F32)
    dnp = dp / (jnp.sqrt(rp + 1e-8) + 1.0)
    outs[0][...] = m
    outs[1][...] = dnp * phip
    if last:
        weo_ref, beo_ref = w[7], w[8]
        outs[2][...] = jnp.dot(m, weo_ref[...], preferred_element_type=F32) + beo_ref[...]


def _tc_edge(first, last, ghr, ghc, e, dxy, wts, e_cnt, hid):
    grid = (e_cnt // BE,)
    blk = lambda r, c: pl.BlockSpec((r, c), lambda i: (i, 0))
    const = lambda r, c: pl.BlockSpec((r, c), lambda i: (0, 0))
    e_w = e.shape[1]
    in_specs = [blk(BE, hid), blk(BE, hid), blk(BE, e_w), blk(BE // 8, hid)]
    in_specs += [const(w.shape[0], w.shape[1]) for w in wts]
    out_specs = [blk(BE, hid), blk(BE // 8, hid)]
    out_shape = [jax.ShapeDtypeStruct((e_cnt, hid), F32),
                 jax.ShapeDtypeStruct((e_cnt // 8, hid), F32)]
    if last:
        out_specs.append(blk(BE, 16))
        out_shape.append(jax.ShapeDtypeStruct((e_cnt, 16), F32))
    body = functools.partial(_edge_body, first, last)
    return pl.pallas_call(
        body, grid=grid, in_specs=in_specs, out_specs=out_specs,
        out_shape=out_shape,
    )(ghr, ghc, e, dxy, *wts)


def _node_body(final, *refs):
    (h_ref, agg_ref, dx_ref, xp_ref,
     wn1h_ref, wn1a0_ref, wn1a1_ref, bn1_ref, wn2_ref, bn2_ref,
     wa_ref, wb_ref, bx_ref, *outs) = refs
    u = _silu(jnp.dot(h_ref[...], wn1h_ref[...], preferred_element_type=F32)
              + jnp.dot(agg_ref[0], wn1a0_ref[...], preferred_element_type=F32)
              + jnp.dot(agg_ref[1], wn1a1_ref[...], preferred_element_type=F32)
              + bn1_ref[...])
    hn = h_ref[...] + jnp.dot(u, wn2_ref[...], preferred_element_type=F32) + bn2_ref[...]
    outs[1][...] = xp_ref[...] + dx_ref[0] + dx_ref[1]
    if final:
        outs[0][...] = jnp.dot(hn, wa_ref[...], preferred_element_type=F32) + bx_ref[...]
    else:
        outs[0][...] = hn
        outs[2][...] = jnp.dot(hn, wa_ref[...], preferred_element_type=F32) + bx_ref[...]
        outs[3][...] = jnp.dot(hn, wb_ref[...], preferred_element_type=F32)


def _tc_node(final, h, agg, dx, xp, wts, n, hid):
    grid = (n // BN,)
    blk = lambda r, c: pl.BlockSpec((r, c), lambda i: (i, 0))
    const = lambda r, c: pl.BlockSpec((r, c), lambda i: (0, 0))
    in_specs = [blk(BN, hid),
                pl.BlockSpec((2, BN, hid // 2), lambda i: (0, i, 0)),
                pl.BlockSpec((2, BN, XP), lambda i: (0, i, 0)),
                blk(BN, XP)]
    in_specs += [const(w.shape[0], w.shape[1]) for w in wts]
    if final:
        out_specs = [blk(BN, hid), blk(BN, XP)]
        out_shape = [jax.ShapeDtypeStruct((n, hid), F32),
                     jax.ShapeDtypeStruct((n, XP), F32)]
    else:
        out_specs = [blk(BN, hid), blk(BN, XP), blk(BN, hid), blk(BN, hid)]
        out_shape = [jax.ShapeDtypeStruct((n, hid), F32),
                     jax.ShapeDtypeStruct((n, XP), F32),
                     jax.ShapeDtypeStruct((n, hid), F32),
                     jax.ShapeDtypeStruct((n, hid), F32)]
    body = functools.partial(_node_body, final)
    return pl.pallas_call(
        body, grid=grid, in_specs=in_specs, out_specs=out_specs,
        out_shape=out_shape,
    )(h, agg, dx, xp, *wts)


def _pipe_gather(idx_h, tab_h, out_h, base, n_chunks, ch,
                 idx2, row2, gs, os_):
    def step(c, buf):
        oth = 1 - buf

        @pl.when(c >= 2)
        def _():
            pltpu.make_async_copy(
                row2.at[buf], out_h.at[pl.ds(base + (c - 2) * ch, ch)],
                os_[buf]).wait()

        pltpu.sync_copy(idx_h.at[pl.ds(base + c * ch, ch)], idx2.at[buf])
        pltpu.async_copy(tab_h.at[idx2.at[buf]], row2.at[buf], gs[buf])

        @pl.when(c >= 1)
        def _():
            pltpu.make_async_copy(tab_h.at[idx2.at[oth]], row2.at[oth],
                                  gs[oth]).wait()
            pltpu.async_copy(row2.at[oth],
                             out_h.at[pl.ds(base + (c - 1) * ch, ch)],
                             os_[oth])

    def body(c, _):
        @pl.when(c % 2 == 0)
        def _():
            step(c, 0)

        @pl.when(c % 2 == 1)
        def _():
            step(c, 1)
        return 0

    lax.fori_loop(0, n_chunks, body, 0)
    lastb = (n_chunks - 1) % 2
    pltpu.make_async_copy(tab_h.at[idx2.at[lastb]], row2.at[lastb],
                          gs[lastb]).wait()
    pltpu.async_copy(row2.at[lastb],
                     out_h.at[pl.ds(base + (n_chunks - 1) * ch, ch)],
                     os_[lastb])
    pltpu.make_async_copy(
        row2.at[1 - lastb], out_h.at[pl.ds(base + (n_chunks - 2) * ch, ch)],
        os_[1 - lastb]).wait()
    pltpu.make_async_copy(
        row2.at[lastb], out_h.at[pl.ds(base + (n_chunks - 1) * ch, ch)],
        os_[lastb]).wait()


def _sc_gather(hw1, hw2, xp, row, col, n, e_cnt, hid):
    info = plsc.get_sparse_core_info()
    nc, ns = info.num_cores, info.num_subcores
    nw = nc * ns
    per_w = e_cnt // nw
    kk = per_w // CH
    mesh = plsc.VectorSubcoreMesh(core_axis_name="c", subcore_axis_name="s")

    kx = per_w // CXD

    @functools.partial(
        pl.kernel, mesh=mesh,
        compiler_params=pltpu.CompilerParams(use_tc_tiling_on_sc=False),
        out_type=(jax.ShapeDtypeStruct((e_cnt, hid), F32),
                  jax.ShapeDtypeStruct((e_cnt, hid), F32),
                  jax.ShapeDtypeStruct((e_cnt // 8, hid), F32)),
        scratch_types=[pltpu.VMEM((2, CH), jnp.int32),
                       pltpu.VMEM((2, CH, hid), F32),
                       pltpu.VMEM((CXD,), jnp.int32),
                       pltpu.VMEM((CXD,), jnp.int32),
                       pltpu.VMEM((CXD, XP), F32),
                       pltpu.VMEM((CXD, XP), F32),
                       pltpu.VMEM((2, CXD // 8, hid), F32),
                       pltpu.SemaphoreType.DMA,
                       pltpu.SemaphoreType.DMA,
                       pltpu.SemaphoreType.DMA,
                       pltpu.SemaphoreType.DMA],
    )
    def k(hw1_h, hw2_h, xp_h, row_h, col_h, ghr_h, ghc_h, dxy_h,
          idx2, r128_2, idxr, idxc, xa, xb, xd2, g0, g1, o0, o1):
        wid = lax.axis_index("s") * nc + lax.axis_index("c")
        base = wid * per_w
        gs, os_ = (g0, g1), (o0, o1)
        _pipe_gather(row_h, hw1_h, ghr_h, base, kk, CH, idx2, r128_2, gs, os_)
        _pipe_gather(col_h, hw2_h, ghc_h, base, kk, CH, idx2, r128_2, gs, os_)

        def dwait(c, buf):
            pltpu.make_async_copy(
                xd2.at[buf],
                dxy_h.at[pl.ds((base + c * CXD) // 8, CXD // 8)],
                os_[buf]).wait()

        def dstep(c, buf):
            @pl.when(c >= 2)
            def _():
                dwait(c - 2, buf)

            off = base + c * CXD
            pltpu.sync_copy(row_h.at[pl.ds(off, CXD)], idxr)
            pltpu.sync_copy(col_h.at[pl.ds(off, CXD)], idxc)
            pltpu.async_copy(xp_h.at[idxr], xa, g0)
            pltpu.async_copy(xp_h.at[idxc], xb, g1)
            pltpu.make_async_copy(xp_h.at[idxr], xa, g0).wait()
            pltpu.make_async_copy(xp_h.at[idxc], xb, g1).wait()

            def sub(j, _):
                xd2[buf, j // 8, pl.ds((j % 8) * XP, XP)] = xa[j] - xb[j]
                return 0
            lax.fori_loop(0, CXD, sub, 0)
            pltpu.async_copy(xd2.at[buf],
                             dxy_h.at[pl.ds(off // 8, CXD // 8)], os_[buf])

        def dbody(c, _):
            @pl.when(c % 2 == 0)
            def _():
                dstep(c, 0)

            @pl.when(c % 2 == 1)
            def _():
                dstep(c, 1)
            return 0

        lax.fori_loop(0, kx, dbody, 0)
        dwait(kx - 2, (kx - 2) % 2)
        dwait(kx - 1, (kx - 1) % 2)

    return k(hw1, hw2, xp, row, col)


def _sc_scatter(m, trans, row, zeros_h, zeros_x, n, e_cnt, hid):
    info = plsc.get_sparse_core_info()
    nc, ns = info.num_cores, info.num_subcores
    hh = hid // nc
    per_w = e_cnt // ns
    half_e = e_cnt // nc
    per_w_t = half_e // ns
    npad = zeros_h.shape[0]
    rows_per_tile = npad // ns
    mesh = plsc.VectorSubcoreMesh(core_axis_name="c", subcore_axis_name="s")

    @functools.partial(
        pl.kernel, mesh=mesh,
        compiler_params=pltpu.CompilerParams(use_tc_tiling_on_sc=False),
        out_type=(jax.ShapeDtypeStruct((nc, npad, hh), F32),
                  jax.ShapeDtypeStruct((nc, npad, XP), F32)),
        scratch_types=[pltpu.VMEM((2, CH), jnp.int32),
                       pltpu.VMEM((2, CH, hh), F32),
                       pltpu.VMEM((2, CH, XP), F32),
                       pltpu.VMEM((2, CH // 8, hid), F32),
                       pltpu.VMEM_SHARED((npad, hh), F32),
                       pltpu.VMEM_SHARED((npad, XP), F32),
                       pltpu.SemaphoreType.DMA,
                       pltpu.SemaphoreType.DMA,
                       pltpu.SemaphoreType.DMA,
                       pltpu.SemaphoreType.DMA],
    )
    def k(m_h, t_h, row_h, z_h, zx_h, agg_h, dx_h,
          idx2, m2, t2, t2p, acc_s, accx_s, l0, l1, s0, s1):
        cid = lax.axis_index("c")
        sid = lax.axis_index("s")
        ls, ss = (l0, l1), (s0, s1)

        @pl.when(sid == 0)
        def _zero():
            pltpu.sync_copy(z_h, acc_s)
            pltpu.sync_copy(zx_h, accx_s)

        plsc.subcore_barrier()

        def add_loop(load_start, load_wait, buf2, acc, base, n_chunks,
                     prep=None):
            def add_wait(buf):
                pltpu.make_async_copy(buf2.at[buf], acc.at[idx2.at[buf]],
                                      ss[buf]).wait()

            def step(c, buf):
                oth = 1 - buf

                @pl.when(c >= 2)
                def _():
                    add_wait(buf)

                pltpu.sync_copy(row_h.at[pl.ds(base + c * CH, CH)],
                                idx2.at[buf])
                load_start(c, buf)

                @pl.when(c >= 1)
                def _():
                    load_wait(c - 1, oth)
                    if prep is not None:
                        prep(oth)
                    pltpu.async_copy(buf2.at[oth], acc.at[idx2.at[oth]],
                                     ss[oth], add=True)

            def body(c, _):
                @pl.when(c % 2 == 0)
                def _():
                    step(c, 0)

                @pl.when(c % 2 == 1)
                def _():
                    step(c, 1)
                return 0

            lax.fori_loop(0, n_chunks, body, 0)
            lastb = (n_chunks - 1) % 2
            load_wait(n_chunks - 1, lastb)
            if prep is not None:
                prep(lastb)
            pltpu.async_copy(buf2.at[lastb], acc.at[idx2.at[lastb]],
                             ss[lastb], add=True)
            add_wait(1 - lastb)
            add_wait(lastb)

        m_base = sid * per_w

        def load_m(c, buf):
            pltpu.async_copy(
                m_h.at[pl.ds(m_base + c * CH, CH), pl.ds(cid * hh, hh)],
                m2.at[buf], ls[buf])

        def wait_m(c, buf):
            pltpu.make_async_copy(
                m_h.at[pl.ds(m_base + c * CH, CH), pl.ds(cid * hh, hh)],
                m2.at[buf], ls[buf]).wait()

        add_loop(load_m, wait_m, m2, acc_s, m_base, per_w // CH)

        t_base = cid * half_e + sid * per_w_t

        def load_t(c, buf):
            pltpu.async_copy(t_h.at[pl.ds((t_base + c * CH) // 8, CH // 8)],
                             t2p.at[buf], ls[buf])

        def wait_t(c, buf):
            pltpu.make_async_copy(
                t_h.at[pl.ds((t_base + c * CH) // 8, CH // 8)],
                t2p.at[buf], ls[buf]).wait()

        def prep_t(buf):
            def up(j, _):
                t2[buf, j] = t2p[buf, j // 8, pl.ds((j % 8) * XP, XP)]
                return 0
            lax.fori_loop(0, CH, up, 0)

        add_loop(load_t, wait_t, t2, accx_s, t_base, per_w_t // CH,
                 prep=prep_t)

        plsc.subcore_barrier()

        r0 = sid * rows_per_tile
        pltpu.sync_copy(acc_s.at[pl.ds(r0, rows_per_tile)],
                        agg_h.at[cid, pl.ds(r0, rows_per_tile)])
        pltpu.sync_copy(accx_s.at[pl.ds(r0, rows_per_tile)],
                        dx_h.at[cid, pl.ds(r0, rows_per_tile)])

    return k(m, trans, row, zeros_h, zeros_x)


def kernel(h, x, edges, edge_attr, params):
    n, in_node = h.shape
    e_cnt = edges.shape[1]
    hid = params['emb_node'][0].shape[1]
    row = edges[0].astype(jnp.int32)
    col = edges[1].astype(jnp.int32)
    xp = jnp.zeros((n, XP), F32).at[:, :3].set(x)
    npad = ((n + 127) // 128) * 128
    zeros_h = jnp.zeros((npad, hid // 2), F32)
    zeros_x = jnp.zeros((npad, XP), F32)

    layers = params['layers']

    def split_w1(lp):
        w1, b1 = lp['edge_mlp1']
        return (w1[:hid], w1[hid:2 * hid], w1[2 * hid:2 * hid + 1],
                w1[2 * hid + 1:], b1.reshape(1, hid))

    w1a0, w1b0, wr0, w1e0, b10 = split_w1(layers[0])
    we, be = params['emb_node']
    wee, bee = params['emb_edge']

    h0, hw1, hw2, wfea, bfea = _tc_init(
        h, we, be.reshape(1, hid), w1a0, w1b0, b10,
        wee, w1e0, bee.reshape(1, hid), n, hid)

    hcur, xpcur = h0, xp
    e_feat = edge_attr
    for li, lp in enumerate(layers):
        first = li == 0
        last = li == len(layers) - 1
        w1a, w1b, wr, w1e, b1 = split_w1(lp)
        w2, b2 = lp['edge_mlp2']
        wc1, bc1 = lp['coord_mlp1']
        wc2, bc2 = lp['coord_mlp2']
        ghr, ghc, dxy = _sc_gather(hw1, hw2, xpcur, row, col, n, e_cnt, hid)
        bc2v = jnp.broadcast_to(bc2.reshape(1, 1), (1, hid))
        if first:
            wts = [wfea, bfea, wr, w2, b2.reshape(1, hid),
                   wc1, bc1.reshape(1, hid), wc2.reshape(1, hid), bc2v]
        else:
            wts = [w1e, wr, w2, b2.reshape(1, hid),
                   wc1, bc1.reshape(1, hid), wc2.reshape(1, hid), bc2v]
        if last:
            weo, beo = params['emb_edge_out']
            wts += [weo, beo.reshape(1, 16)]
        eouts = _tc_edge(first, last, ghr, ghc, e_feat, dxy, wts, e_cnt, hid)
        m = eouts[0]
        trans = eouts[1]
        agg, dx = _sc_scatter(m, trans, row, zeros_h, zeros_x, n, e_cnt, hid)
        wn1, bn1 = lp['node_mlp1']
        wn2, bn2 = lp['node_mlp2']
        hh = hid // 2
        if last:
            who, bho = params['emb_node_out']
            nwts = [wn1[:hid], wn1[hid:hid + hh], wn1[hid + hh:], bn1.reshape(1, hid),
                    wn2, bn2.reshape(1, hid), who, who, bho.reshape(1, in_node)]
            hout, xpcur = _tc_node(True, hcur, agg, dx, xpcur, nwts, n, hid)
        else:
            w1a_n, w1b_n, _, _, b1_n = split_w1(layers[li + 1])
            nwts = [wn1[:hid], wn1[hid:hid + hh], wn1[hid + hh:], bn1.reshape(1, hid),
                    wn2, bn2.reshape(1, hid), w1a_n, w1b_n, b1_n]
            hcur, xpcur, hw1, hw2 = _tc_node(False, hcur, agg, dx, xpcur, nwts, n, hid)
        e_feat = m

    e_out = eouts[2]
    return (hout, xpcur[:, :3], e_out)


# speedup vs baseline: 1.0637x; 1.0637x over previous
import functools

import jax
import jax.numpy as jnp
from jax import lax
from jax.experimental import pallas as pl
from jax.experimental.pallas import tpu as pltpu
from jax.experimental.pallas import tpu_sc as plsc

F32 = jnp.float32
BF16 = jnp.bfloat16
XP = 16
BE = 2560
BN = 2000
CH = 400
CXD = 200


def _silu(v):
    return v * jax.nn.sigmoid(v)


def _init_body(h_ref, we_ref, be_ref, w1a_ref, w1b_ref, b1_ref,
               wee_ref, w1e_ref, bee_ref,
               h0_ref, hw1_ref, hw2_ref, wfea_ref, bfea_ref):
    h0 = jnp.dot(h_ref[...], we_ref[...], preferred_element_type=F32) + be_ref[...]
    h0_ref[...] = h0
    hw1_ref[...] = jnp.dot(h0, w1a_ref[...], preferred_element_type=F32) + b1_ref[...]
    hw2_ref[...] = jnp.dot(h0, w1b_ref[...], preferred_element_type=F32)
    wfea_ref[...] = jnp.dot(wee_ref[...], w1e_ref[...], preferred_element_type=F32)
    bfea_ref[...] = jnp.dot(bee_ref[...], w1e_ref[...], preferred_element_type=F32)


def _tc_init(h, we, be, w1a, w1b, b1, wee, w1e, bee, n, hid):
    grid = (n // BN,)
    blk = lambda r, c: pl.BlockSpec((r, c), lambda i: (i, 0))
    const = lambda r, c: pl.BlockSpec((r, c), lambda i: (0, 0))
    return pl.pallas_call(
        _init_body,
        grid=grid,
        in_specs=[blk(BN, hid), const(hid, hid), const(1, hid),
                  const(hid, hid), const(hid, hid), const(1, hid),
                  const(16, hid), const(hid, hid), const(1, hid)],
        out_specs=[blk(BN, hid), blk(BN, hid), blk(BN, hid),
                   const(16, hid), const(1, hid)],
        out_shape=[jax.ShapeDtypeStruct((n, hid), F32),
                   jax.ShapeDtypeStruct((n, hid), F32),
                   jax.ShapeDtypeStruct((n, hid), F32),
                   jax.ShapeDtypeStruct((16, hid), F32),
                   jax.ShapeDtypeStruct((1, hid), F32)],
    )(h, we, be, w1a, w1b, b1, wee, w1e, bee)


def _edge_body(first, last, *refs):
    n_wts = 4 + (9 if first else 8) + (2 if last else 0)
    ins, outs = refs[:n_wts], refs[n_wts:]
    ghr_ref, ghc_ref, e_ref, dxy_ref = ins[:4]
    w = list(ins[4:])
    if first:
        w1e_ref, bfea_ref = w[0], w[1]
        w = w[2:]
    else:
        w1e_ref = w[0]
        w = w[1:]
    wr_ref, w2_ref, b2_ref, wc1_ref, bc1_ref, wc2_ref, bc2_ref = w[:7]
    hid = ghr_ref.shape[1]
    be8 = dxy_ref.shape[0]
    dp = dxy_ref[...]
    li = lax.broadcasted_iota(jnp.int32, (hid, hid), 0)
    lj = lax.broadcasted_iota(jnp.int32, (hid, hid), 1)
    gb = (li // XP == lj // XP).astype(F32)
    rp = jnp.dot(dp * dp, gb, preferred_element_type=F32)
    s_a = lax.broadcasted_iota(jnp.int32, (hid, 8), 0)
    s_g = lax.broadcasted_iota(jnp.int32, (hid, 8), 1)
    sel = (s_a == s_g * XP).astype(F32)
    r8 = jnp.dot(rp, sel, preferred_element_type=F32)
    ne = 8 * be8
    a_j = lax.broadcasted_iota(jnp.int32, (ne, 8), 0)
    a_g = lax.broadcasted_iota(jnp.int32, (ne, 8), 1)
    a8 = (a_j % 8 == a_g).astype(F32)
    rrep = lax.broadcast_in_dim(r8, (be8, 8, 8), (0, 2)).reshape(ne, 8)
    radial = jnp.sum(rrep * a8, axis=1, keepdims=True)
    z = (ghr_ref[...] + ghc_ref[...]
         + radial * wr_ref[...]
         + jnp.dot(e_ref[...], w1e_ref[...], preferred_element_type=F32))
    if first:
        z = z + bfea_ref[...]
    m1 = _silu(z)
    m = _silu(jnp.dot(m1, w2_ref[...], preferred_element_type=F32) + b2_ref[...])
    c1 = _silu(jnp.dot(m, wc1_ref[...], preferred_element_type=F32) + bc1_ref[...])
    phi = jnp.sum(c1 * wc2_ref[...], axis=1, keepdims=True) + bc2_ref[:, :1]
    b_g = lax.broadcasted_iota(jnp.int32, (8, hid), 0)
    b_l = lax.broadcasted_iota(jnp.int32, (8, hid), 1)
    b8 = (b_l // XP == b_g).astype(F32)
    phi8 = jnp.sum((phi * a8).reshape(be8, 8, 8), axis=1)
    phip = jnp.dot(phi8, b8, preferred_element_type=F32)
    dnp = dp / (jnp.sqrt(rp + 1e-8) + 1.0)
    outs[0][...] = m
    outs[1][...] = dnp * phip
    if last:
        weo_ref, beo_ref = w[7], w[8]
        outs[2][...] = jnp.dot(m, weo_ref[...], preferred_element_type=F32) + beo_ref[...]


def _tc_edge(first, last, ghr, ghc, e, dxy, wts, e_cnt, hid):
    grid = (e_cnt // BE,)
    blk = lambda r, c: pl.BlockSpec((r, c), lambda i: (i, 0))
    const = lambda r, c: pl.BlockSpec((r, c), lambda i: (0, 0))
    e_w = e.shape[1]
    in_specs = [blk(BE, hid), blk(BE, hid), blk(BE, e_w), blk(BE // 8, hid)]
    in_specs += [const(w.shape[0], w.shape[1]) for w in wts]
    out_specs = [blk(BE, hid), blk(BE // 8, hid)]
    out_shape = [jax.ShapeDtypeStruct((e_cnt, hid), F32),
                 jax.ShapeDtypeStruct((e_cnt // 8, hid), F32)]
    if last:
        out_specs.append(blk(BE, 16))
        out_shape.append(jax.ShapeDtypeStruct((e_cnt, 16), F32))
    body = functools.partial(_edge_body, first, last)
    return pl.pallas_call(
        body, grid=grid, in_specs=in_specs, out_specs=out_specs,
        out_shape=out_shape,
    )(ghr, ghc, e, dxy, *wts)


def _node_body(final, *refs):
    (h_ref, agg_ref, dx_ref, xp_ref,
     wn1h_ref, wn1a0_ref, wn1a1_ref, bn1_ref, wn2_ref, bn2_ref,
     wa_ref, wb_ref, bx_ref, *outs) = refs
    u = _silu(jnp.dot(h_ref[...], wn1h_ref[...], preferred_element_type=F32)
              + jnp.dot(agg_ref[0], wn1a0_ref[...], preferred_element_type=F32)
              + jnp.dot(agg_ref[1], wn1a1_ref[...], preferred_element_type=F32)
              + bn1_ref[...])
    hn = h_ref[...] + jnp.dot(u, wn2_ref[...], preferred_element_type=F32) + bn2_ref[...]
    outs[1][...] = xp_ref[...] + dx_ref[0] + dx_ref[1]
    if final:
        outs[0][...] = jnp.dot(hn, wa_ref[...], preferred_element_type=F32) + bx_ref[...]
    else:
        outs[0][...] = hn
        outs[2][...] = jnp.dot(hn, wa_ref[...], preferred_element_type=F32) + bx_ref[...]
        outs[3][...] = jnp.dot(hn, wb_ref[...], preferred_element_type=F32)


def _tc_node(final, h, agg, dx, xp, wts, n, hid):
    grid = (n // BN,)
    blk = lambda r, c: pl.BlockSpec((r, c), lambda i: (i, 0))
    const = lambda r, c: pl.BlockSpec((r, c), lambda i: (0, 0))
    in_specs = [blk(BN, hid),
                pl.BlockSpec((2, BN, hid // 2), lambda i: (0, i, 0)),
                pl.BlockSpec((2, BN, XP), lambda i: (0, i, 0)),
                blk(BN, XP)]
    in_specs += [const(w.shape[0], w.shape[1]) for w in wts]
    if final:
        out_specs = [blk(BN, hid), blk(BN, XP)]
        out_shape = [jax.ShapeDtypeStruct((n, hid), F32),
                     jax.ShapeDtypeStruct((n, XP), F32)]
    else:
        out_specs = [blk(BN, hid), blk(BN, XP), blk(BN, hid), blk(BN, hid)]
        out_shape = [jax.ShapeDtypeStruct((n, hid), F32),
                     jax.ShapeDtypeStruct((n, XP), F32),
                     jax.ShapeDtypeStruct((n, hid), F32),
                     jax.ShapeDtypeStruct((n, hid), F32)]
    body = functools.partial(_node_body, final)
    return pl.pallas_call(
        body, grid=grid, in_specs=in_specs, out_specs=out_specs,
        out_shape=out_shape,
    )(h, agg, dx, xp, *wts)


def _pipe_gather(idx_h, tab_h, out_h, base, n_chunks, ch,
                 idx2, row2, gs, os_):
    def step(c, buf):
        oth = 1 - buf

        @pl.when(c >= 2)
        def _():
            pltpu.make_async_copy(
                row2.at[buf], out_h.at[pl.ds(base + (c - 2) * ch, ch)],
                os_[buf]).wait()

        pltpu.sync_copy(idx_h.at[pl.ds(base + c * ch, ch)], idx2.at[buf])
        pltpu.async_copy(tab_h.at[idx2.at[buf]], row2.at[buf], gs[buf])

        @pl.when(c >= 1)
        def _():
            pltpu.make_async_copy(tab_h.at[idx2.at[oth]], row2.at[oth],
                                  gs[oth]).wait()
            pltpu.async_copy(row2.at[oth],
                             out_h.at[pl.ds(base + (c - 1) * ch, ch)],
                             os_[oth])

    def body(c, _):
        @pl.when(c % 2 == 0)
        def _():
            step(c, 0)

        @pl.when(c % 2 == 1)
        def _():
            step(c, 1)
        return 0

    lax.fori_loop(0, n_chunks, body, 0)
    lastb = (n_chunks - 1) % 2
    pltpu.make_async_copy(tab_h.at[idx2.at[lastb]], row2.at[lastb],
                          gs[lastb]).wait()
    pltpu.async_copy(row2.at[lastb],
                     out_h.at[pl.ds(base + (n_chunks - 1) * ch, ch)],
                     os_[lastb])
    pltpu.make_async_copy(
        row2.at[1 - lastb], out_h.at[pl.ds(base + (n_chunks - 2) * ch, ch)],
        os_[1 - lastb]).wait()
    pltpu.make_async_copy(
        row2.at[lastb], out_h.at[pl.ds(base + (n_chunks - 1) * ch, ch)],
        os_[lastb]).wait()


def _sc_gather(hw1, hw2, xp, row, col, n, e_cnt, hid):
    info = plsc.get_sparse_core_info()
    nc, ns = info.num_cores, info.num_subcores
    nw = nc * ns
    per_w = e_cnt // nw
    kk = per_w // CH
    mesh = plsc.VectorSubcoreMesh(core_axis_name="c", subcore_axis_name="s")

    kx = per_w // CXD

    @functools.partial(
        pl.kernel, mesh=mesh,
        compiler_params=pltpu.CompilerParams(use_tc_tiling_on_sc=False),
        out_type=(jax.ShapeDtypeStruct((e_cnt, hid), F32),
                  jax.ShapeDtypeStruct((e_cnt, hid), F32),
                  jax.ShapeDtypeStruct((e_cnt // 8, hid), F32)),
        scratch_types=[pltpu.VMEM((2, CH), jnp.int32),
                       pltpu.VMEM((2, CH, hid), F32),
                       pltpu.VMEM((CXD,), jnp.int32),
                       pltpu.VMEM((CXD,), jnp.int32),
                       pltpu.VMEM((CXD, XP), F32),
                       pltpu.VMEM((CXD, XP), F32),
                       pltpu.VMEM((2, CXD // 8, hid), F32),
                       pltpu.SemaphoreType.DMA,
                       pltpu.SemaphoreType.DMA,
                       pltpu.SemaphoreType.DMA,
                       pltpu.SemaphoreType.DMA],
    )
    def k(hw1_h, hw2_h, xp_h, row_h, col_h, ghr_h, ghc_h, dxy_h,
          idx2, r128_2, idxr, idxc, xa, xb, xd2, g0, g1, o0, o1):
        wid = lax.axis_index("s") * nc + lax.axis_index("c")
        base = wid * per_w
        gs, os_ = (g0, g1), (o0, o1)
        _pipe_gather(row_h, hw1_h, ghr_h, base, kk, CH, idx2, r128_2, gs, os_)
        _pipe_gather(col_h, hw2_h, ghc_h, base, kk, CH, idx2, r128_2, gs, os_)

        def dwait(c, buf):
            pltpu.make_async_copy(
                xd2.at[buf],
                dxy_h.at[pl.ds((base + c * CXD) // 8, CXD // 8)],
                os_[buf]).wait()

        def dstep(c, buf):
            @pl.when(c >= 2)
            def _():
                dwait(c - 2, buf)

            off = base + c * CXD
            pltpu.sync_copy(row_h.at[pl.ds(off, CXD)], idxr)
            pltpu.sync_copy(col_h.at[pl.ds(off, CXD)], idxc)
            pltpu.async_copy(xp_h.at[idxr], xa, g0)
            pltpu.async_copy(xp_h.at[idxc], xb, g1)
            pltpu.make_async_copy(xp_h.at[idxr], xa, g0).wait()
            pltpu.make_async_copy(xp_h.at[idxc], xb, g1).wait()

            @plsc.parallel_loop(0, CXD, unroll=8)
            def _sub(j):
                xd2[buf, j // 8, pl.ds((j % 8) * XP, XP)] = xa[j] - xb[j]
            pltpu.async_copy(xd2.at[buf],
                             dxy_h.at[pl.ds(off // 8, CXD // 8)], os_[buf])

        def dbody(c, _):
            @pl.when(c % 2 == 0)
            def _():
                dstep(c, 0)

            @pl.when(c % 2 == 1)
            def _():
                dstep(c, 1)
            return 0

        lax.fori_loop(0, kx, dbody, 0)
        dwait(kx - 2, (kx - 2) % 2)
        dwait(kx - 1, (kx - 1) % 2)

    return k(hw1, hw2, xp, row, col)


def _sc_scatter(m, trans, row, zeros_h, zeros_x, n, e_cnt, hid):
    info = plsc.get_sparse_core_info()
    nc, ns = info.num_cores, info.num_subcores
    hh = hid // nc
    per_w = e_cnt // ns
    half_e = e_cnt // nc
    per_w_t = half_e // ns
    npad = zeros_h.shape[0]
    rows_per_tile = npad // ns
    mesh = plsc.VectorSubcoreMesh(core_axis_name="c", subcore_axis_name="s")

    @functools.partial(
        pl.kernel, mesh=mesh,
        compiler_params=pltpu.CompilerParams(use_tc_tiling_on_sc=False),
        out_type=(jax.ShapeDtypeStruct((nc, npad, hh), F32),
                  jax.ShapeDtypeStruct((nc, npad, XP), F32)),
        scratch_types=[pltpu.VMEM((2, CH), jnp.int32),
                       pltpu.VMEM((2, CH, hh), F32),
                       pltpu.VMEM((2, CH, XP), F32),
                       pltpu.VMEM((2, CH // 8, hid), F32),
                       pltpu.VMEM_SHARED((npad, hh), F32),
                       pltpu.VMEM_SHARED((npad, XP), F32),
                       pltpu.SemaphoreType.DMA,
                       pltpu.SemaphoreType.DMA,
                       pltpu.SemaphoreType.DMA,
                       pltpu.SemaphoreType.DMA],
    )
    def k(m_h, t_h, row_h, z_h, zx_h, agg_h, dx_h,
          idx2, m2, t2, t2p, acc_s, accx_s, l0, l1, s0, s1):
        cid = lax.axis_index("c")
        sid = lax.axis_index("s")
        ls, ss = (l0, l1), (s0, s1)

        @pl.when(sid == 0)
        def _zero():
            pltpu.sync_copy(z_h, acc_s)
            pltpu.sync_copy(zx_h, accx_s)

        plsc.subcore_barrier()

        def add_loop(load_start, load_wait, buf2, acc, base, n_chunks,
                     prep=None):
            def add_wait(buf):
                pltpu.make_async_copy(buf2.at[buf], acc.at[idx2.at[buf]],
                                      ss[buf]).wait()

            def step(c, buf):
                oth = 1 - buf

                @pl.when(c >= 2)
                def _():
                    add_wait(buf)

                pltpu.sync_copy(row_h.at[pl.ds(base + c * CH, CH)],
                                idx2.at[buf])
                load_start(c, buf)

                @pl.when(c >= 1)
                def _():
                    load_wait(c - 1, oth)
                    if prep is not None:
                        prep(oth)
                    pltpu.async_copy(buf2.at[oth], acc.at[idx2.at[oth]],
                                     ss[oth], add=True)

            def body(c, _):
                @pl.when(c % 2 == 0)
                def _():
                    step(c, 0)

                @pl.when(c % 2 == 1)
                def _():
                    step(c, 1)
                return 0

            lax.fori_loop(0, n_chunks, body, 0)
            lastb = (n_chunks - 1) % 2
            load_wait(n_chunks - 1, lastb)
            if prep is not None:
                prep(lastb)
            pltpu.async_copy(buf2.at[lastb], acc.at[idx2.at[lastb]],
                             ss[lastb], add=True)
            add_wait(1 - lastb)
            add_wait(lastb)

        m_base = sid * per_w

        def load_m(c, buf):
            pltpu.async_copy(
                m_h.at[pl.ds(m_base + c * CH, CH), pl.ds(cid * hh, hh)],
                m2.at[buf], ls[buf])

        def wait_m(c, buf):
            pltpu.make_async_copy(
                m_h.at[pl.ds(m_base + c * CH, CH), pl.ds(cid * hh, hh)],
                m2.at[buf], ls[buf]).wait()

        add_loop(load_m, wait_m, m2, acc_s, m_base, per_w // CH)

        t_base = cid * half_e + sid * per_w_t

        def load_t(c, buf):
            pltpu.async_copy(t_h.at[pl.ds((t_base + c * CH) // 8, CH // 8)],
                             t2p.at[buf], ls[buf])

        def wait_t(c, buf):
            pltpu.make_async_copy(
                t_h.at[pl.ds((t_base + c * CH) // 8, CH // 8)],
                t2p.at[buf], ls[buf]).wait()

        def prep_t(buf):
            @plsc.parallel_loop(0, CH, unroll=8)
            def _up(j):
                t2[buf, j] = t2p[buf, j // 8, pl.ds((j % 8) * XP, XP)]

        add_loop(load_t, wait_t, t2, accx_s, t_base, per_w_t // CH,
                 prep=prep_t)

        plsc.subcore_barrier()

        r0 = sid * rows_per_tile
        pltpu.sync_copy(acc_s.at[pl.ds(r0, rows_per_tile)],
                        agg_h.at[cid, pl.ds(r0, rows_per_tile)])
        pltpu.sync_copy(accx_s.at[pl.ds(r0, rows_per_tile)],
                        dx_h.at[cid, pl.ds(r0, rows_per_tile)])

    return k(m, trans, row, zeros_h, zeros_x)


def kernel(h, x, edges, edge_attr, params):
    n, in_node = h.shape
    e_cnt = edges.shape[1]
    hid = params['emb_node'][0].shape[1]
    row = edges[0].astype(jnp.int32)
    col = edges[1].astype(jnp.int32)
    xp = jnp.zeros((n, XP), F32).at[:, :3].set(x)
    npad = ((n + 127) // 128) * 128
    zeros_h = jnp.zeros((npad, hid // 2), F32)
    zeros_x = jnp.zeros((npad, XP), F32)

    layers = params['layers']

    def split_w1(lp):
        w1, b1 = lp['edge_mlp1']
        return (w1[:hid], w1[hid:2 * hid], w1[2 * hid:2 * hid + 1],
                w1[2 * hid + 1:], b1.reshape(1, hid))

    w1a0, w1b0, wr0, w1e0, b10 = split_w1(layers[0])
    we, be = params['emb_node']
    wee, bee = params['emb_edge']

    h0, hw1, hw2, wfea, bfea = _tc_init(
        h, we, be.reshape(1, hid), w1a0, w1b0, b10,
        wee, w1e0, bee.reshape(1, hid), n, hid)

    hcur, xpcur = h0, xp
    e_feat = edge_attr
    for li, lp in enumerate(layers):
        first = li == 0
        last = li == len(layers) - 1
        w1a, w1b, wr, w1e, b1 = split_w1(lp)
        w2, b2 = lp['edge_mlp2']
        wc1, bc1 = lp['coord_mlp1']
        wc2, bc2 = lp['coord_mlp2']
        ghr, ghc, dxy = _sc_gather(hw1, hw2, xpcur, row, col, n, e_cnt, hid)
        bc2v = jnp.broadcast_to(bc2.reshape(1, 1), (1, hid))
        if first:
            wts = [wfea, bfea, wr, w2, b2.reshape(1, hid),
                   wc1, bc1.reshape(1, hid), wc2.reshape(1, hid), bc2v]
        else:
            wts = [w1e, wr, w2, b2.reshape(1, hid),
                   wc1, bc1.reshape(1, hid), wc2.reshape(1, hid), bc2v]
        if last:
            weo, beo = params['emb_edge_out']
            wts += [weo, beo.reshape(1, 16)]
        eouts = _tc_edge(first, last, ghr, ghc, e_feat, dxy, wts, e_cnt, hid)
        m = eouts[0]
        trans = eouts[1]
        agg, dx = _sc_scatter(m, trans, row, zeros_h, zeros_x, n, e_cnt, hid)
        wn1, bn1 = lp['node_mlp1']
        wn2, bn2 = lp['node_mlp2']
        hh = hid // 2
        if last:
            who, bho = params['emb_node_out']
            nwts = [wn1[:hid], wn1[hid:hid + hh], wn1[hid + hh:], bn1.reshape(1, hid),
                    wn2, bn2.reshape(1, hid), who, who, bho.reshape(1, in_node)]
            hout, xpcur = _tc_node(True, hcur, agg, dx, xpcur, nwts, n, hid)
        else:
            w1a_n, w1b_n, _, _, b1_n = split_w1(layers[li + 1])
            nwts = [wn1[:hid], wn1[hid:hid + hh], wn1[hid + hh:], bn1.reshape(1, hid),
                    wn2, bn2.reshape(1, hid), w1a_n, w1b_n, b1_n]
            hcur, xpcur, hw1, hw2 = _tc_node(False, hcur, agg, dx, xpcur, nwts, n, hid)
        e_feat = m

    e_out = eouts[2]
    return (hout, xpcur[:, :3], e_out)
